# Initial kernel scaffold; baseline (speedup 1.0000x reference)
import functools
import jax
import jax.numpy as jnp
from jax import lax
from jax.experimental import pallas as pl
from jax.experimental.pallas import tpu as pltpu, tpu_sc as plsc

_VOCAB = 1000000
_EMB = 32
_B = 16384
_L = 50

_NC = 2
_NS = 16
_NW = _NC * _NS
_BPW = _B // _NW
_NB = 32
_NBATCH = _BPW // _NB


def _sc_kernel(centers_hbm, ctx_hbm, in_hbm, out_emb_hbm, pred_hbm,
               idxc_v, idxu_v, v_v, u_v, o_v, sem_v, sem_u):
    wid = lax.axis_index("s") * _NC + lax.axis_index("c")
    u_rows = u_v.reshape(_NB * _L, _EMB)
    n_idx = _NB * _L

    def batch_body(t, carry):
        base = wid * _BPW + t * _NB
        pltpu.sync_copy(ctx_hbm.at[pl.ds(base * _L, n_idx)], idxu_v)
        pltpu.sync_copy(centers_hbm.at[pl.ds(base, _NB)], idxc_v)
        cp_v = pltpu.async_copy(in_hbm.at[idxc_v], v_v, sem_v)
        cp_u = pltpu.async_copy(out_emb_hbm.at[idxu_v], u_rows, sem_u)
        cp_v.wait()
        cp_u.wait()

        def b_body(b, carry2):
            boff = b * (_L * _EMB)
            acc0 = jnp.zeros((16,), jnp.float32)
            acc1 = jnp.zeros((16,), jnp.float32)
            acc2 = jnp.zeros((16,), jnp.float32)
            acc3 = jnp.zeros((16,), jnp.float32)
            bvec = jnp.full((16,), b, jnp.int32)
            for e in range(_EMB):
                w = plsc.load_gather(
                    v_v, [bvec, jnp.full((16,), e, jnp.int32)])
                off = boff + e * _L
                acc0 = acc0 + w * u_v[pl.ds(off, 16)]
                acc1 = acc1 + w * u_v[pl.ds(off + 16, 16)]
                acc2 = acc2 + w * u_v[pl.ds(off + 32, 16)]
                acc3 = acc3 + w * u_v[pl.ds(off + 34, 16)]
            oo = b * _L
            o_v[pl.ds(oo, 16)] = acc0
            o_v[pl.ds(oo + 16, 16)] = acc1
            o_v[pl.ds(oo + 32, 16)] = acc2
            o_v[pl.ds(oo + 34, 16)] = acc3
            return carry2

        lax.fori_loop(0, _NB, b_body, 0)
        pltpu.sync_copy(o_v, pred_hbm.at[pl.ds(base * _L, _NB * _L)])
        return carry

    lax.fori_loop(0, _NBATCH, batch_body, 0)


@jax.jit
def _run(centers_flat, ctx_flat, in_emb, out_emb):
    mesh = plsc.VectorSubcoreMesh(core_axis_name="c", subcore_axis_name="s")
    f = pl.kernel(
        _sc_kernel,
        out_type=jax.ShapeDtypeStruct((_B * _L,), jnp.float32),
        mesh=mesh,
        scratch_types=[
            pltpu.VMEM((_NB,), jnp.int32),
            pltpu.VMEM((_NB * _L,), jnp.int32),
            pltpu.VMEM((_NB, _EMB), jnp.float32),
            pltpu.VMEM((_NB * _L * _EMB,), jnp.float32),
            pltpu.VMEM((_NB * _L,), jnp.float32),
            pltpu.SemaphoreType.DMA,
            pltpu.SemaphoreType.DMA,
        ],
    )
    return f(centers_flat, ctx_flat, in_emb, out_emb)


def kernel(centers, contexts_negatives, in_emb, out_emb):
    centers_flat = centers.reshape(_B).astype(jnp.int32)
    ctx_flat = contexts_negatives.reshape(_B * _L).astype(jnp.int32)
    pred = _run(centers_flat, ctx_flat, in_emb, out_emb)
    return pred.reshape(_B, 1, _L)


# speedup vs baseline: 1.9059x; 1.9059x over previous
import functools
import jax
import jax.numpy as jnp
from jax import lax
from jax.experimental import pallas as pl
from jax.experimental.pallas import tpu as pltpu, tpu_sc as plsc

_VOCAB = 1000000
_EMB = 32
_B = 16384
_L = 50

_NC = 2
_NS = 16
_NW = _NC * _NS
_BPW = _B // _NW
_NB = 32
_NBATCH = _BPW // _NB
_JOFF = (0, 16, 32, 34)


def _sc_kernel(centers_hbm, ctx_hbm, in_hbm, out_emb_hbm, pred_hbm,
               idxc_v, idxu_v, v_v, u_v, o_v, sem_v, sem_u):
    wid = lax.axis_index("s") * _NC + lax.axis_index("c")
    n_idx = _NB * _L

    def batch_body(t, carry):
        base = wid * _BPW + t * _NB
        pltpu.sync_copy(ctx_hbm.at[pl.ds(base * _L, n_idx)], idxu_v)
        pltpu.sync_copy(centers_hbm.at[pl.ds(base, _NB)], idxc_v)
        cp_v = pltpu.async_copy(in_hbm.at[idxc_v], v_v, sem_v)
        cp_u = pltpu.async_copy(out_emb_hbm.at[idxu_v], u_v, sem_u)
        cp_v.wait()
        cp_u.wait()

        def b_body(b, carry2):
            fbase = b * (_L * _EMB) + lax.broadcasted_iota(
                jnp.int32, (16,), 0)
            bvec = jnp.full((16,), b, jnp.int32)
            accs = [jnp.zeros((16,), jnp.float32) for _ in range(4)]
            for e in range(_EMB):
                w = plsc.load_gather(
                    v_v, [bvec, jnp.full((16,), e, jnp.int32)])
                for j, joff in enumerate(_JOFF):
                    f = fbase + (e * _L + joff)
                    x = plsc.load_gather(
                        u_v,
                        [lax.shift_right_logical(f, 5),
                         lax.bitwise_and(f, 31)])
                    accs[j] = accs[j] + w * x
            oo = b * _L
            for j, joff in enumerate(_JOFF):
                o_v[pl.ds(oo + joff, 16)] = accs[j]
            return carry2

        lax.fori_loop(0, _NB, b_body, 0)
        pltpu.sync_copy(o_v, pred_hbm.at[pl.ds(base * _L, _NB * _L)])
        return carry

    lax.fori_loop(0, _NBATCH, batch_body, 0)


@jax.jit
def _run(centers_flat, ctx_flat, in_emb, out_emb):
    mesh = plsc.VectorSubcoreMesh(core_axis_name="c", subcore_axis_name="s")
    f = pl.kernel(
        _sc_kernel,
        out_type=jax.ShapeDtypeStruct((_B * _L,), jnp.float32),
        mesh=mesh,
        scratch_types=[
            pltpu.VMEM((_NB,), jnp.int32),
            pltpu.VMEM((_NB * _L,), jnp.int32),
            pltpu.VMEM((_NB, _EMB), jnp.float32),
            pltpu.VMEM((_NB * _L, _EMB), jnp.float32),
            pltpu.VMEM((_NB * _L,), jnp.float32),
            pltpu.SemaphoreType.DMA,
            pltpu.SemaphoreType.DMA,
        ],
        compiler_params=pltpu.CompilerParams(
            use_tc_tiling_on_sc=False, needs_layout_passes=False),
    )
    return f(centers_flat, ctx_flat, in_emb, out_emb)


def kernel(centers, contexts_negatives, in_emb, out_emb):
    centers_flat = centers.reshape(_B).astype(jnp.int32)
    ctx_flat = contexts_negatives.reshape(_B * _L).astype(jnp.int32)
    pred = _run(centers_flat, ctx_flat, in_emb, out_emb)
    return pred.reshape(_B, 1, _L)
